# Initial kernel scaffold; baseline (speedup 1.0000x reference)
import dataclasses
import functools

import jax
import jax.numpy as jnp
from jax import lax
from jax.experimental import pallas as pl
from jax.experimental.pallas import tpu as pltpu
from jax.experimental.pallas import tpu_sc as plsc

B, N, FD = 4, 50000, 128
NK = N // 2
NPAD = 50176
NG = NPAD // 128
BINS = 2048
PIPE = 8
TN = 2000
TS = 1000
CH = 200
MINT = jnp.int32(-(2 ** 31))

_CP_SC = pltpu.CompilerParams()
if "needs_layout_passes" in pltpu.CompilerParams.__dataclass_fields__:
    _CP_SC = dataclasses.replace(_CP_SC, needs_layout_passes=False)

_mesh = plsc.VectorSubcoreMesh(core_axis_name="c", subcore_axis_name="s")


def _proj_body(h_ref, wt_ref, b_ref, key_ref):
    x = h_ref[0]
    w = jnp.sum(x * wt_ref[...], axis=-1) + b_ref[0]
    u = lax.bitcast_convert_type(w, jnp.int32)
    m = lax.shift_right_arithmetic(u, 31)
    key = jnp.bitwise_not(jnp.bitwise_xor(u, jnp.bitwise_or(m, MINT)))
    key_ref[...] = key[None]


def _proj(h, wt, b):
    return pl.pallas_call(
        _proj_body,
        grid=(B, N // TN),
        in_specs=[
            pl.BlockSpec((1, TN, FD), lambda i, j: (i, j, 0)),
            pl.BlockSpec((1, FD), lambda i, j: (0, 0)),
            pl.BlockSpec(memory_space=pltpu.SMEM),
        ],
        out_specs=pl.BlockSpec((1, TN), lambda i, j: (i, j)),
        out_shape=jax.ShapeDtypeStruct((B, N), jnp.int32),
    )(h, wt, b)


def _sort_body(keys_hbm, ks_hbm, is_hbm, keyv, idxv, hist, posbuf, drainv,
               spk, spi, sem):
    c = lax.axis_index("c")
    s = lax.axis_index("s")

    @pl.when(s < 2)
    def _():
        b = c * 2 + s
        pltpu.sync_copy(keys_hbm.at[b], keyv.at[pl.ds(0, N)])
        iota = lax.broadcasted_iota(jnp.int32, (16,), 0)

        @pl.loop(N // 16, NPAD // 16)
        def _(j):
            keyv[pl.ds(j * 16, 16)] = jnp.full((16,), -1, jnp.int32)

        base0 = b * N

        @pl.loop(0, NPAD // 16)
        def _(j):
            idxv[pl.ds(j * 16, 16)] = iota + (base0 + j * 16)

        def drain_one():
            pltpu.make_async_copy(
                keys_hbm.at[b, pl.ds(0, 128)], drainv, sem).wait()

        for shift, mask in ((0, 2047), (11, 2047), (22, 1023)):
            is_last = shift == 22

            @pl.loop(0, BINS // 16)
            def _(j):
                hist[pl.ds(j * 16, 16)] = jnp.zeros((16,), jnp.int32)

            @pl.loop(0, NPAD // 16)
            def _(j):
                d = lax.shift_right_logical(keyv[pl.ds(j * 16, 16)], shift) & mask
                cnt, last = plsc.scan_count(d)
                plsc.addupdate_scatter(hist, [d], cnt, mask=last)

            def _scan(j, carry):
                v = hist[pl.ds(j * 16, 16)]
                inc = plsc.cumsum(v)
                hist[pl.ds(j * 16, 16)] = inc - v + carry
                return carry + jnp.sum(v)

            lax.fori_loop(0, BINS // 16, _scan, s * NPAD)

            @pl.loop(0, NG, step=PIPE)
            def _(g0):
                for r in range(PIPE):
                    @pl.when(g0 > 0)
                    def _():
                        drain_one()
                        drain_one()
                    g = g0 + r
                    for j in range(8):
                        sl = pl.ds(g * 128 + j * 16, 16)
                        d = lax.shift_right_logical(keyv[sl], shift) & mask
                        cnt, last = plsc.scan_count(d)
                        base = plsc.load_gather(hist, [d])
                        posbuf[r, pl.ds(j * 16, 16)] = base + cnt - 1
                        plsc.addupdate_scatter(hist, [d], cnt, mask=last)
                    src = pl.ds(g * 128, 128)
                    pltpu.async_copy(keyv.at[src], spk.at[posbuf.at[r]], sem)
                    pltpu.async_copy(idxv.at[src], spi.at[posbuf.at[r]], sem)

            @pl.loop(0, 2 * PIPE)
            def _(t):
                drain_one()

            sp_slice = pl.ds(s * NPAD, NPAD)
            if not is_last:
                pltpu.sync_copy(spk.at[sp_slice], keyv)
                pltpu.sync_copy(spi.at[sp_slice], idxv)
            else:
                pltpu.sync_copy(spk.at[sp_slice], ks_hbm.at[b])
                pltpu.sync_copy(spi.at[sp_slice], is_hbm.at[b])


def _sort(keys):
    kern = pl.kernel(
        _sort_body,
        out_type=(jax.ShapeDtypeStruct((B, NPAD), jnp.int32),
                  jax.ShapeDtypeStruct((B, NPAD), jnp.int32)),
        mesh=_mesh,
        compiler_params=_CP_SC,
        scratch_types=[
            pltpu.VMEM((NPAD,), jnp.int32),
            pltpu.VMEM((NPAD,), jnp.int32),
            pltpu.VMEM((BINS,), jnp.int32),
            pltpu.VMEM((PIPE, 128), jnp.int32),
            pltpu.VMEM((128,), jnp.int32),
            pltpu.VMEM_SHARED((2 * NPAD,), jnp.int32),
            pltpu.VMEM_SHARED((2 * NPAD,), jnp.int32),
            pltpu.SemaphoreType.DMA,
        ],
    )
    return kern(keys)


def _gather_body(hflat_hbm, is_hbm, rows_hbm, idxc, rowbuf, sem):
    c = lax.axis_index("c")
    s = lax.axis_index("s")
    wid = s * 2 + c
    b = wid // 8
    t = wid % 8
    nchunks = NK // CH

    @pl.loop(0, 16)
    def _(i):
        ck = t + 8 * i

        @pl.when(ck < nchunks)
        def _():
            pltpu.sync_copy(is_hbm.at[b, pl.ds(ck * CH, CH)], idxc)
            pltpu.async_copy(hflat_hbm.at[idxc], rowbuf, sem).wait()
            pltpu.sync_copy(rowbuf, rows_hbm.at[b, pl.ds(ck * CH, CH), :])


def _gather(hflat, isrt):
    kern = pl.kernel(
        _gather_body,
        out_type=jax.ShapeDtypeStruct((B, NK, FD), jnp.float32),
        mesh=_mesh,
        compiler_params=_CP_SC,
        scratch_types=[
            pltpu.VMEM((CH,), jnp.int32),
            pltpu.VMEM((CH, FD), jnp.float32),
            pltpu.SemaphoreType.DMA,
        ],
    )
    return kern(hflat, isrt)


def _scale_body(rows_ref, keys_ref, out_ref):
    k = keys_ref[...]
    asc = jnp.bitwise_not(k)
    mm = lax.shift_right_arithmetic(asc, 31)
    xmask = jnp.bitwise_or(MINT, jnp.bitwise_and(jnp.bitwise_not(mm),
                                                 jnp.int32(0x7FFFFFFF)))
    w = lax.bitcast_convert_type(jnp.bitwise_xor(asc, xmask), jnp.float32)
    sc = jax.nn.sigmoid(w)
    out_ref[...] = rows_ref[...] * sc[:, :, None]


def _scale(rows, ks):
    return pl.pallas_call(
        _scale_body,
        grid=(B, NK // TS),
        in_specs=[
            pl.BlockSpec((1, TS, FD), lambda i, j: (i, j, 0)),
            pl.BlockSpec((1, TS), lambda i, j: (i, j)),
        ],
        out_specs=pl.BlockSpec((1, TS, FD), lambda i, j: (i, j, 0)),
        out_shape=jax.ShapeDtypeStruct((B, NK, FD), jnp.float32),
    )(rows, ks)


@jax.jit
def kernel(h, W, b):
    wt = W.reshape(1, FD)
    keys = _proj(h, wt, b)
    ks, isrt = _sort(keys)
    rows = _gather(h.reshape(B * N, FD), isrt)
    return _scale(rows, ks)


# speedup vs baseline: 2.3002x; 2.3002x over previous
import dataclasses

import jax
import jax.numpy as jnp
from jax import lax
from jax.experimental import pallas as pl
from jax.experimental.pallas import tpu as pltpu
from jax.experimental.pallas import tpu_sc as plsc

B, N, FD = 4, 50000, 128
NK = N // 2
NPAD = 50176
NG = NPAD // 128
BINS = 2048
PIPE = 8
TN = 2000
TS = 1000
CH = 256
MINT = -(2 ** 31)

_CP_SC = pltpu.CompilerParams()
if "needs_layout_passes" in pltpu.CompilerParams.__dataclass_fields__:
    _CP_SC = dataclasses.replace(_CP_SC, needs_layout_passes=False)

_mesh = plsc.VectorSubcoreMesh(core_axis_name="c", subcore_axis_name="s")


def _proj_body(h_ref, w_ref, b_ref, key_ref):
    x = h_ref[0]
    wv = jnp.dot(x, w_ref[...], preferred_element_type=jnp.float32)
    s = jax.nn.sigmoid(wv[:, 0] + b_ref[0])
    u = lax.bitcast_convert_type(s, jnp.int32)
    m = lax.shift_right_arithmetic(u, 31)
    key = jnp.bitwise_not(jnp.bitwise_xor(u, jnp.bitwise_or(m, jnp.int32(MINT))))
    key_ref[...] = key[None, None]


def _proj(h, W, b):
    return pl.pallas_call(
        _proj_body,
        grid=(B, N // TN),
        in_specs=[
            pl.BlockSpec((1, TN, FD), lambda i, j: (i, j, 0)),
            pl.BlockSpec((FD, 1), lambda i, j: (0, 0)),
            pl.BlockSpec(memory_space=pltpu.SMEM),
        ],
        out_specs=pl.BlockSpec((1, 1, TN), lambda i, j: (i * (N // TN) + j, 0, 0)),
        out_shape=jax.ShapeDtypeStruct((B * N // TN, 1, TN), jnp.int32),
    )(h, W, b).reshape(B, N)


def _sort_body(keys_hbm, ks_hbm, is_hbm, keyv, idxv, hist, posbuf, drainv,
               spk, spi, sem):
    c = lax.axis_index("c")
    s = lax.axis_index("s")

    @pl.when(s < 2)
    def _():
        b = c * 2 + s
        pltpu.sync_copy(keys_hbm.at[b], keyv)
        iota = lax.broadcasted_iota(jnp.int32, (16,), 0)
        base0 = b * N

        @pl.loop(0, NPAD // 16)
        def _(j):
            idxv[pl.ds(j * 16, 16)] = iota + (base0 + j * 16)

        def drain_one():
            pltpu.make_async_copy(
                keys_hbm.at[b, pl.ds(0, 128)], drainv, sem).wait()

        for shift, mask in ((0, 2047), (11, 2047), (22, 1023)):
            is_last = shift == 22

            @pl.loop(0, BINS // 16)
            def _(j):
                hist[pl.ds(j * 16, 16)] = jnp.zeros((16,), jnp.int32)

            @pl.loop(0, NPAD // 16)
            def _(j):
                d = lax.shift_right_logical(keyv[pl.ds(j * 16, 16)], shift) & mask
                cnt, last = plsc.scan_count(d)
                plsc.addupdate_scatter(hist, [d], cnt, mask=last)

            def _scan(j, carry):
                v = hist[pl.ds(j * 16, 16)]
                inc = plsc.cumsum(v)
                hist[pl.ds(j * 16, 16)] = inc - v + carry
                return carry + jnp.sum(v)

            lax.fori_loop(0, BINS // 16, _scan, s * NPAD)

            @pl.loop(0, NG, step=PIPE)
            def _(g0):
                for r in range(PIPE):
                    @pl.when(g0 > 0)
                    def _():
                        drain_one()
                        drain_one()
                    g = g0 + r
                    for j in range(8):
                        sl = pl.ds(g * 128 + j * 16, 16)
                        d = lax.shift_right_logical(keyv[sl], shift) & mask
                        cnt, last = plsc.scan_count(d)
                        base = plsc.load_gather(hist, [d])
                        posbuf[r, pl.ds(j * 16, 16)] = base + cnt - 1
                        plsc.addupdate_scatter(hist, [d], cnt, mask=last)
                    src = pl.ds(g * 128, 128)
                    pltpu.async_copy(keyv.at[src], spk.at[posbuf.at[r]], sem)
                    pltpu.async_copy(idxv.at[src], spi.at[posbuf.at[r]], sem)

            @pl.loop(0, 2 * PIPE)
            def _(t):
                drain_one()

            sp_slice = pl.ds(s * NPAD, NPAD)
            if not is_last:
                pltpu.sync_copy(spk.at[sp_slice], keyv)
                pltpu.sync_copy(spi.at[sp_slice], idxv)
            else:
                pltpu.sync_copy(spk.at[sp_slice], ks_hbm.at[b])
                pltpu.sync_copy(spi.at[sp_slice], is_hbm.at[b])


def _sort(keys):
    kern = pl.kernel(
        _sort_body,
        out_type=(jax.ShapeDtypeStruct((B, NPAD), jnp.int32),
                  jax.ShapeDtypeStruct((B, NPAD), jnp.int32)),
        mesh=_mesh,
        compiler_params=_CP_SC,
        scratch_types=[
            pltpu.VMEM((NPAD,), jnp.int32),
            pltpu.VMEM((NPAD,), jnp.int32),
            pltpu.VMEM((BINS,), jnp.int32),
            pltpu.VMEM((PIPE, 128), jnp.int32),
            pltpu.VMEM((128,), jnp.int32),
            pltpu.VMEM_SHARED((2 * NPAD,), jnp.int32),
            pltpu.VMEM_SHARED((2 * NPAD,), jnp.int32),
            pltpu.SemaphoreType.DMA,
        ],
    )
    return kern(keys)


def _gather_body(hflat_hbm, is_hbm, rows_hbm, idxc, rowbuf, sem):
    c = lax.axis_index("c")
    s = lax.axis_index("s")
    wid = s * 2 + c
    b = wid // 8
    t = wid % 8
    nchunks = -(-NK // CH)
    tail = NK - (nchunks - 1) * CH

    @pl.loop(0, 13)
    def _(i):
        ck = t + 8 * i

        @pl.when(ck < nchunks)
        def _():
            pltpu.sync_copy(is_hbm.at[b, pl.ds(ck * CH, CH)], idxc)
            pltpu.async_copy(hflat_hbm.at[idxc], rowbuf, sem).wait()

            @pl.when(ck < nchunks - 1)
            def _():
                pltpu.sync_copy(rowbuf, rows_hbm.at[b, pl.ds(ck * CH, CH), :])

            @pl.when(ck == nchunks - 1)
            def _():
                pltpu.sync_copy(
                    rowbuf.at[pl.ds(0, tail)],
                    rows_hbm.at[b, pl.ds(ck * CH, tail), :])


def _gather(hflat, isrt):
    kern = pl.kernel(
        _gather_body,
        out_type=jax.ShapeDtypeStruct((B, NK, FD), jnp.float32),
        mesh=_mesh,
        compiler_params=_CP_SC,
        scratch_types=[
            pltpu.VMEM((CH,), jnp.int32),
            pltpu.VMEM((CH, FD), jnp.float32),
            pltpu.SemaphoreType.DMA,
        ],
    )
    return kern(hflat, isrt)


def _scale_body(rows_ref, keys_ref, out_ref):
    k = keys_ref[0]
    asc = jnp.bitwise_not(k)
    mm = lax.shift_right_arithmetic(asc, 31)
    xmask = jnp.bitwise_or(jnp.int32(MINT), jnp.bitwise_and(jnp.bitwise_not(mm),
                                                 jnp.int32(0x7FFFFFFF)))
    sc = lax.bitcast_convert_type(jnp.bitwise_xor(asc, xmask), jnp.float32)
    out_ref[...] = rows_ref[...] * sc[0][None, :, None]


def _scale(rows, ks):
    return pl.pallas_call(
        _scale_body,
        grid=(B, NK // TS),
        in_specs=[
            pl.BlockSpec((1, TS, FD), lambda i, j: (i, j, 0)),
            pl.BlockSpec((1, 1, TS), lambda i, j: (i * (NK // TS) + j, 0, 0)),
        ],
        out_specs=pl.BlockSpec((1, TS, FD), lambda i, j: (i, j, 0)),
        out_shape=jax.ShapeDtypeStruct((B, NK, FD), jnp.float32),
    )(rows, ks.reshape(B * NK // TS, 1, TS))


@jax.jit
def kernel(h, W, b):
    keys = jnp.pad(_proj(h, W, b), ((0, 0), (0, NPAD - N)),
                   constant_values=-1)
    ks, isrt = _sort(keys)
    rows = _gather(h.reshape(B * N, FD), isrt)
    return _scale(rows, lax.slice(ks, (0, 0), (B, NK)))
